# baseline (speedup 1.0000x reference)
import functools
import jax
import jax.numpy as jnp
from jax.experimental import pallas as pl
from jax.experimental.pallas import tpu as pltpu

_P = 0.5


def _linear_block(h_ref, w_ref, b_ref, o_ref):
    o_ref[...] = (
        jnp.dot(h_ref[...], w_ref[...], preferred_element_type=jnp.float32)
        + b_ref[...]
    )


def _linear(h, W_lin, b_lin):
    n, d = h.shape
    blk = 400
    grid = n // blk
    return pl.pallas_call(
        _linear_block,
        grid=(grid,),
        in_specs=[
            pl.BlockSpec((blk, d), lambda i: (i, 0)),
            pl.BlockSpec((d, d), lambda i: (0, 0)),
            pl.BlockSpec((1, d), lambda i: (0, 0)),
        ],
        out_specs=pl.BlockSpec((blk, d), lambda i: (i, 0)),
        out_shape=jax.ShapeDtypeStruct((n, d), jnp.float32),
    )(h, W_lin.T, b_lin[None, :])


def kernel(x, W_mlp, b_mlp, W_lin, b_lin, edge_index):
    N = x.shape[0]
    E = edge_index.shape[1]
    src = edge_index[0]
    dst = edge_index[1]
    h_mlp = jnp.tanh(x @ W_mlp.T + b_mlp)
    d = jnp.sum(jnp.abs(h_mlp[src] - h_mlp[dst]), axis=1)
    order = jnp.lexsort((d, dst))
    dst_s = dst[order]
    src_s = src[order]
    deg = jnp.bincount(dst, length=N)
    starts = jnp.cumsum(deg) - deg
    rank = jnp.arange(E) - starts[dst_s]
    keep = jnp.ceil(deg.astype(jnp.float32) * _P).astype(jnp.int32)
    mask = rank < keep[dst_s]
    msg = jnp.where(mask[:, None], x[src_s], 0.0)
    agg_sum = jax.ops.segment_sum(msg, dst_s, num_segments=N)
    cnt = jnp.maximum(keep, 1).astype(jnp.float32)[:, None]
    h_agg = agg_sum / cnt
    h_homo = _P * h_agg + x
    return _linear(h_homo, W_lin, b_lin)


# speedup vs baseline: 9.9567x; 9.9567x over previous
import functools

import jax
import jax.numpy as jnp
from jax import lax
from jax.experimental import pallas as pl
from jax.experimental.pallas import tpu as pltpu
from jax.experimental.pallas import tpu_sc as plsc

_P = 0.5
_N = 10000
_NPAD = 10240
_E = 320000
_D = 128
_NW = 16
_CH = 128
_NCH = 160
_EPW = _NCH * _CH
_EPAD = _NW * _EPW
_NODES_PW = _NPAD // _NW
_NV = _NODES_PW // 16
_HIST = 16 * _NPAD
_DUMP = _HIST
_NPASS = 8

_mesh = plsc.VectorSubcoreMesh(
    core_axis_name="c", subcore_axis_name="s", num_cores=1)
_params = pltpu.CompilerParams(needs_layout_passes=False)


def _mlp_body(x_ref, w_ref, b_ref, o_ref):
    o_ref[...] = jnp.tanh(
        jnp.dot(x_ref[...], w_ref[...], preferred_element_type=jnp.float32)
        + b_ref[...])


def _tc_mlp(x_pad, w_big, b_big):
    blk = 256
    return pl.pallas_call(
        _mlp_body,
        grid=(_NPAD // blk,),
        in_specs=[
            pl.BlockSpec((blk, _D), lambda i: (i, 0)),
            pl.BlockSpec((_D, _D), lambda i: (0, 0)),
            pl.BlockSpec((1, _D), lambda i: (0, 0)),
        ],
        out_specs=pl.BlockSpec((blk, _D), lambda i: (i, 0)),
        out_shape=jax.ShapeDtypeStruct((_NPAD, _D), jnp.float32),
    )(x_pad, w_big, b_big)


def _out_body(a_ref, x_ref, w_ref, b_ref, o_ref):
    o_ref[...] = (
        jnp.dot(a_ref[...] + x_ref[...], w_ref[...],
                preferred_element_type=jnp.float32)
        + b_ref[...])


def _tc_out(acc, x_pad, w_t, b_row):
    blk = 256
    return pl.pallas_call(
        _out_body,
        grid=(_NPAD // blk,),
        in_specs=[
            pl.BlockSpec((blk, _D), lambda i: (i, 0)),
            pl.BlockSpec((blk, _D), lambda i: (i, 0)),
            pl.BlockSpec((_D, _D), lambda i: (0, 0)),
            pl.BlockSpec((1, _D), lambda i: (0, 0)),
        ],
        out_specs=pl.BlockSpec((blk, _D), lambda i: (i, 0)),
        out_shape=jax.ShapeDtypeStruct((_NPAD, _D), jnp.float32),
    )(acc, x_pad, w_t, b_row)


def _select_body(src_hbm, dst_hbm, h_hbm, w_out, invc_out,
                 src_v, dst_v, key_v, hw_v, pref_v, frac_v,
                 hstage, need_v, prefst_v, fracst_v, invf_v,
                 zero_v, ones_v, sidx,
                 hist_sh, pref_sh, frac_sh, sem):
    wid = lax.axis_index("s")
    nbase = wid * _NODES_PW
    ebase = pl.multiple_of(wid * _EPW, 8)
    lane = lax.iota(jnp.int32, 16)

    pltpu.sync_copy(src_hbm.at[pl.ds(ebase, _EPW)], src_v)
    pltpu.sync_copy(dst_hbm.at[pl.ds(ebase, _EPW)], dst_v)
    pltpu.sync_copy(h_hbm, hw_v)
    for i in range(_CH // 16):
        ones_v[pl.ds(i * 16, 16)] = jnp.ones((16,), jnp.int32)
    for i in range(_NODES_PW // 16):
        zero_v[pl.ds(i * 16, 16)] = jnp.zeros((16,), jnp.int32)

    @pl.loop(0, _NCH)
    def _(c):
        for v in range(8):
            e = pl.ds(c * _CH + v * 16, 16)
            s = src_v[e]
            t = dst_v[e]
            h0s = plsc.load_gather(hw_v, [2 * s])
            h1s = plsc.load_gather(hw_v, [2 * s + 1])
            h0d = plsc.load_gather(hw_v, [2 * t])
            h1d = plsc.load_gather(hw_v, [2 * t + 1])
            d = jnp.abs(h0s - h0d) + jnp.abs(h1s - h1d)
            key_v[e] = plsc.bitcast(d, jnp.int32)

    for p in range(_NPASS):
        shift_lo = 28 - 4 * p
        for dgt in range(16):
            pltpu.sync_copy(
                zero_v,
                hist_sh.at[pl.ds(
                    pl.multiple_of(dgt * _NPAD + nbase, 8), _NODES_PW)])
        plsc.subcore_barrier()

        @pl.loop(0, _NCH)
        def _(c):
            slot = lax.rem(c, 8)
            for v in range(8):
                e = pl.ds(c * _CH + v * 16, 16)
                t = dst_v[e]
                k = key_v[e]
                dig = (k >> shift_lo) & 15
                tgt = dig * _NPAD + t
                if p > 0:
                    pr = plsc.load_gather(pref_v, [t])
                    act = (k >> (shift_lo + 4)) == pr
                    dump = _DUMP + ((v + c) & 7) * 16 + lane
                    tgt = jnp.where(act, tgt, dump)
                sidx[slot, pl.ds(v * 16, 16)] = tgt

            @pl.when(c >= 8)
            def _():
                pltpu.make_async_copy(
                    ones_v, hist_sh.at[sidx.at[slot]], sem).wait()
            pltpu.async_copy(
                ones_v, hist_sh.at[sidx.at[slot]], sem, add=True)

        @pl.loop(0, 8)
        def _(slot):
            pltpu.make_async_copy(
                ones_v, hist_sh.at[sidx.at[slot]], sem).wait()

        plsc.subcore_barrier()

        for dgt in range(16):
            pltpu.sync_copy(
                hist_sh.at[pl.ds(
                    pl.multiple_of(dgt * _NPAD + nbase, 8), _NODES_PW)],
                hstage.at[dgt])

        @pl.loop(0, _NV)
        def _(j):
            sl = pl.ds(j * 16, 16)
            if p == 0:
                deg = jnp.zeros((16,), jnp.int32)
                for dgt in range(16):
                    deg = deg + hstage[dgt, sl]
                keep = (deg + 1) >> 1
                need = keep
                invf_v[sl] = 1.0 / jnp.maximum(keep, 1).astype(jnp.float32)
            else:
                need = need_v[sl]
            cum = jnp.zeros((16,), jnp.int32)
            done = need <= 0
            sel = jnp.zeros((16,), jnp.int32)
            cumb = jnp.zeros((16,), jnp.int32)
            heq = jnp.ones((16,), jnp.int32)
            for dgt in range(16):
                h = hstage[dgt, sl]
                nc = cum + h
                cross = jnp.logical_and(jnp.logical_not(done), nc >= need)
                sel = jnp.where(cross, dgt, sel)
                cumb = jnp.where(cross, cum, cumb)
                heq = jnp.where(cross, h, heq)
                done = jnp.logical_or(done, cross)
                cum = nc
            newneed = need - cumb
            need_v[sl] = newneed
            if p == 0:
                newpref = sel
            else:
                oldpref = pref_v[pl.ds(nbase + j * 16, 16)]
                newpref = (oldpref << 4) | sel
            prefst_v[sl] = newpref
            if p == _NPASS - 1:
                frac = newneed.astype(jnp.float32) / heq.astype(jnp.float32)
                fracst_v[sl] = jnp.where(done, frac, 0.0)

        pltpu.sync_copy(
            prefst_v,
            pref_sh.at[pl.ds(pl.multiple_of(nbase, 8), _NODES_PW)])
        if p == 0:
            pltpu.sync_copy(
                invf_v,
                invc_out.at[pl.ds(pl.multiple_of(nbase, 8), _NODES_PW)])
        if p == _NPASS - 1:
            pltpu.sync_copy(
                fracst_v,
                frac_sh.at[pl.ds(pl.multiple_of(nbase, 8), _NODES_PW)])
        plsc.subcore_barrier()
        pltpu.sync_copy(pref_sh, pref_v)
        if p == _NPASS - 1:
            pltpu.sync_copy(frac_sh, frac_v)

    @pl.loop(0, _NCH)
    def _(c):
        for v in range(8):
            e = pl.ds(c * _CH + v * 16, 16)
            t = dst_v[e]
            k = key_v[e]
            thr = plsc.load_gather(pref_v, [t])
            f = plsc.load_gather(frac_v, [t])
            w = jnp.where(k < thr, 1.0, jnp.where(k == thr, f, 0.0))
            hw_v[e] = w
    pltpu.sync_copy(hw_v, w_out.at[pl.ds(ebase, _EPW)])


def _aggregate_body(src_hbm, dst_hbm, w_hbm, x_hbm, invc_hbm, acc_out,
                    srcr, dstr, wr, row_buf, idx_buf, invc_v,
                    acc_sh, stsem, gsem, ssem):
    wid = lax.axis_index("s")
    nbase = wid * _NODES_PW
    ebase = pl.multiple_of(wid * _EPW, 8)
    lane = lax.iota(jnp.int32, 16)

    @pl.loop(0, _CH)
    def _(r):
        for q in range(8):
            row_buf[0, r, pl.ds(q * 16, 16)] = jnp.zeros((16,), jnp.float32)

    @pl.loop(0, _NODES_PW // _CH)
    def _(b):
        pltpu.sync_copy(row_buf.at[0],
                        acc_sh.at[pl.ds(nbase + b * _CH, _CH)])

    @pl.when(wid == 0)
    def _():
        pltpu.sync_copy(row_buf.at[0, pl.ds(0, 16)],
                        acc_sh.at[pl.ds(_NPAD, 16)])
    plsc.subcore_barrier()

    def stage(c):
        r = lax.rem(c, 4)
        off = pl.ds(ebase + c * _CH, _CH)
        pltpu.async_copy(src_hbm.at[off], srcr.at[r], stsem)
        pltpu.async_copy(dst_hbm.at[off], dstr.at[r], stsem)
        pltpu.async_copy(w_hbm.at[off], wr.at[r], stsem)

    def wait_stage(c):
        r = lax.rem(c, 4)
        off = pl.ds(ebase + c * _CH, _CH)
        pltpu.make_async_copy(src_hbm.at[off], srcr.at[r], stsem).wait()
        pltpu.make_async_copy(dst_hbm.at[off], dstr.at[r], stsem).wait()
        pltpu.make_async_copy(w_hbm.at[off], wr.at[r], stsem).wait()

    stage(0)
    stage(1)
    wait_stage(0)
    pltpu.async_copy(x_hbm.at[srcr.at[0]], row_buf.at[0], gsem)

    @pl.loop(0, _NCH)
    def _(c):
        b = lax.rem(c, 2)
        r = lax.rem(c, 4)

        @pl.when(c + 2 < _NCH)
        def _():
            stage(c + 2)

        @pl.when(c >= 1)
        def _():
            bo = lax.rem(c - 1, 2)
            pltpu.make_async_copy(
                row_buf.at[bo], acc_sh.at[idx_buf.at[bo]], ssem).wait()

        @pl.when(c + 1 < _NCH)
        def _():
            rn = lax.rem(c + 1, 4)
            bn = lax.rem(c + 1, 2)
            wait_stage(c + 1)
            pltpu.async_copy(x_hbm.at[srcr.at[rn]], row_buf.at[bn], gsem)

        pltpu.make_async_copy(
            x_hbm.at[srcr.at[r]], row_buf.at[b], gsem).wait()

        for v in range(8):
            e = pl.ds(v * 16, 16)
            wv = wr[r, e]
            dv = dstr[r, e]
            tgt = jnp.where(wv > 0.0, dv, _NPAD + lane)
            idx_buf[b, e] = tgt
            isfrac = jnp.logical_and(wv > 0.0, wv < 1.0)
            nfrac = jnp.max(jnp.where(isfrac, 1, 0))

            @pl.when(nfrac > 0)
            def _():
                @pl.loop(0, 16)
                def _(l):
                    wl = jnp.max(jnp.where(lane == l, wv, 0.0))

                    @pl.when(jnp.logical_and(wl > 0.0, wl < 1.0))
                    def _():
                        for q in range(8):
                            row_buf[b, v * 16 + l, pl.ds(q * 16, 16)] = (
                                row_buf[b, v * 16 + l, pl.ds(q * 16, 16)]
                                * wl)

        pltpu.async_copy(
            row_buf.at[b], acc_sh.at[idx_buf.at[b]], ssem, add=True)

    bo = lax.rem(_NCH - 1, 2)
    pltpu.make_async_copy(
        row_buf.at[bo], acc_sh.at[idx_buf.at[bo]], ssem).wait()
    plsc.subcore_barrier()

    pltpu.sync_copy(
        invc_hbm.at[pl.ds(pl.multiple_of(nbase, 8), _NODES_PW)], invc_v)

    @pl.loop(0, _NODES_PW // _CH)
    def _(b):
        pltpu.sync_copy(acc_sh.at[pl.ds(nbase + b * _CH, _CH)],
                        row_buf.at[0])

        @pl.loop(0, _CH // 16)
        def _(g):
            iv = invc_v[pl.ds(b * _CH + g * 16, 16)] * _P
            for r in range(16):
                s = iv[r]
                for q in range(8):
                    row_buf[0, g * 16 + r, pl.ds(q * 16, 16)] = (
                        row_buf[0, g * 16 + r, pl.ds(q * 16, 16)] * s)
        pltpu.sync_copy(
            row_buf.at[0],
            acc_out.at[pl.ds(pl.multiple_of(nbase + b * _CH, 8), _CH)])


_sc_select = functools.partial(
    pl.kernel,
    out_type=[
        jax.ShapeDtypeStruct((_EPAD,), jnp.float32),
        jax.ShapeDtypeStruct((_NPAD,), jnp.float32),
    ],
    mesh=_mesh,
    compiler_params=_params,
    scratch_types=[
        pltpu.VMEM((_EPW,), jnp.int32),
        pltpu.VMEM((_EPW,), jnp.int32),
        pltpu.VMEM((_EPW,), jnp.int32),
        pltpu.VMEM((2 * _NPAD,), jnp.float32),
        pltpu.VMEM((_NPAD,), jnp.int32),
        pltpu.VMEM((_NPAD,), jnp.float32),
        pltpu.VMEM((16, _NODES_PW), jnp.int32),
        pltpu.VMEM((_NODES_PW,), jnp.int32),
        pltpu.VMEM((_NODES_PW,), jnp.int32),
        pltpu.VMEM((_NODES_PW,), jnp.float32),
        pltpu.VMEM((_NODES_PW,), jnp.float32),
        pltpu.VMEM((_NODES_PW,), jnp.int32),
        pltpu.VMEM((_CH,), jnp.int32),
        pltpu.VMEM((8, _CH), jnp.int32),
        pltpu.VMEM_SHARED((_HIST + 128,), jnp.int32),
        pltpu.VMEM_SHARED((_NPAD,), jnp.int32),
        pltpu.VMEM_SHARED((_NPAD,), jnp.float32),
        pltpu.SemaphoreType.DMA,
    ],
)(_select_body)

_sc_aggregate = functools.partial(
    pl.kernel,
    out_type=[jax.ShapeDtypeStruct((_NPAD, _D), jnp.float32)],
    mesh=_mesh,
    compiler_params=_params,
    scratch_types=[
        pltpu.VMEM((4, _CH), jnp.int32),
        pltpu.VMEM((4, _CH), jnp.int32),
        pltpu.VMEM((4, _CH), jnp.float32),
        pltpu.VMEM((2, _CH, _D), jnp.float32),
        pltpu.VMEM((2, _CH), jnp.int32),
        pltpu.VMEM((_NODES_PW,), jnp.float32),
        pltpu.VMEM_SHARED((_NPAD + 16, _D), jnp.float32),
        pltpu.SemaphoreType.DMA,
        pltpu.SemaphoreType.DMA,
        pltpu.SemaphoreType.DMA,
    ],
)(_aggregate_body)


def kernel(x, W_mlp, b_mlp, W_lin, b_lin, edge_index):
    x_pad = jnp.pad(x, ((0, _NPAD - _N), (0, 0)))
    w_big = jnp.zeros((_D, _D), jnp.float32).at[:, :2].set(W_mlp.T)
    b_big = jnp.zeros((_D,), jnp.float32).at[:2].set(b_mlp)
    h_full = _tc_mlp(x_pad, w_big, b_big[None, :])
    h_flat = h_full[:, :2].reshape(-1)

    src = edge_index[0]
    dst = edge_index[1]
    pad = _EPAD - _E
    src_p = jnp.concatenate([src, jnp.zeros((pad,), jnp.int32)])
    dst_p = jnp.concatenate([dst, jnp.full((pad,), _NPAD - 1, jnp.int32)])

    w_e, invc = _sc_select(src_p, dst_p, h_flat)
    (acc,) = _sc_aggregate(src_p, dst_p, w_e, x, invc)
    out_pad = _tc_out(acc, x_pad, W_lin.T, b_lin[None, :])
    return out_pad[:_N]


# speedup vs baseline: 10.1286x; 1.0173x over previous
import functools

import jax
import jax.numpy as jnp
from jax import lax
from jax.experimental import pallas as pl
from jax.experimental.pallas import tpu as pltpu
from jax.experimental.pallas import tpu_sc as plsc

_P = 0.5
_N = 10000
_NPAD = 10240
_E = 320000
_D = 128
_NW = 16
_CH = 128
_NCH = 160
_EPW = _NCH * _CH
_EPAD = _NW * _EPW
_NODES_PW = _NPAD // _NW
_NV = _NODES_PW // 16
_HIST = 16 * _NPAD
_DUMP = _HIST
_NPASS = 8
_CCH = 96
_EPWC = 20608

_mesh = plsc.VectorSubcoreMesh(
    core_axis_name="c", subcore_axis_name="s", num_cores=1)
_params = pltpu.CompilerParams(needs_layout_passes=False)


def _mlp_body(x_ref, w_ref, b_ref, o_ref):
    o_ref[...] = jnp.tanh(
        jnp.dot(x_ref[...], w_ref[...], preferred_element_type=jnp.float32)
        + b_ref[...])


def _tc_mlp(x_pad, w_big, b_big):
    blk = 256
    return pl.pallas_call(
        _mlp_body,
        grid=(_NPAD // blk,),
        in_specs=[
            pl.BlockSpec((blk, _D), lambda i: (i, 0)),
            pl.BlockSpec((_D, _D), lambda i: (0, 0)),
            pl.BlockSpec((1, _D), lambda i: (0, 0)),
        ],
        out_specs=pl.BlockSpec((blk, _D), lambda i: (i, 0)),
        out_shape=jax.ShapeDtypeStruct((_NPAD, _D), jnp.float32),
    )(x_pad, w_big, b_big)


def _out_body(a_ref, x_ref, w_ref, b_ref, o_ref):
    o_ref[...] = (
        jnp.dot(a_ref[...] + x_ref[...], w_ref[...],
                preferred_element_type=jnp.float32)
        + b_ref[...])


def _tc_out(acc, x_pad, w_t, b_row):
    blk = 256
    return pl.pallas_call(
        _out_body,
        grid=(_NPAD // blk,),
        in_specs=[
            pl.BlockSpec((blk, _D), lambda i: (i, 0)),
            pl.BlockSpec((blk, _D), lambda i: (i, 0)),
            pl.BlockSpec((_D, _D), lambda i: (0, 0)),
            pl.BlockSpec((1, _D), lambda i: (0, 0)),
        ],
        out_specs=pl.BlockSpec((blk, _D), lambda i: (i, 0)),
        out_shape=jax.ShapeDtypeStruct((_NPAD, _D), jnp.float32),
    )(acc, x_pad, w_t, b_row)


def _select_body(src_hbm, dst_hbm, h_hbm, cpk_out, cnts_out, invc_out,
                 frac_out,
                 src_v, dst_v, key_v, hw_v, pref_v, frac_v,
                 hstage, need_v, prefst_v, fracst_v, invf_v,
                 zero_v, ones_v, sidx, cstage,
                 hist_sh, pref_sh, frac_sh, sem):
    wid = lax.axis_index("s")
    nbase = wid * _NODES_PW
    ebase = pl.multiple_of(wid * _EPW, 8)
    lane = lax.iota(jnp.int32, 16)

    pltpu.sync_copy(src_hbm.at[pl.ds(ebase, _EPW)], src_v)
    pltpu.sync_copy(dst_hbm.at[pl.ds(ebase, _EPW)], dst_v)
    pltpu.sync_copy(h_hbm, hw_v.at[pl.ds(0, 2 * _NPAD)])
    for i in range(_CH // 16):
        ones_v[pl.ds(i * 16, 16)] = jnp.ones((16,), jnp.int32)
    for i in range(_NODES_PW // 16):
        zero_v[pl.ds(i * 16, 16)] = jnp.zeros((16,), jnp.int32)

    @pl.loop(0, _NCH)
    def _(c):
        for v in range(8):
            e = pl.ds(c * _CH + v * 16, 16)
            s = src_v[e]
            t = dst_v[e]
            h0s = plsc.load_gather(hw_v, [2 * s])
            h1s = plsc.load_gather(hw_v, [2 * s + 1])
            h0d = plsc.load_gather(hw_v, [2 * t])
            h1d = plsc.load_gather(hw_v, [2 * t + 1])
            d = jnp.abs(h0s - h0d) + jnp.abs(h1s - h1d)
            key_v[e] = plsc.bitcast(d, jnp.int32)

    for p in range(_NPASS):
        shift_lo = 28 - 4 * p
        for dgt in range(16):
            pltpu.sync_copy(
                zero_v,
                hist_sh.at[pl.ds(
                    pl.multiple_of(dgt * _NPAD + nbase, 8), _NODES_PW)])
        plsc.subcore_barrier()

        @pl.loop(0, _NCH)
        def _(c):
            slot = lax.rem(c, 8)
            for v in range(8):
                e = pl.ds(c * _CH + v * 16, 16)
                t = dst_v[e]
                k = key_v[e]
                dig = (k >> shift_lo) & 15
                tgt = dig * _NPAD + t
                if p > 0:
                    pr = plsc.load_gather(pref_v, [t])
                    act = (k >> (shift_lo + 4)) == pr
                    dump = _DUMP + ((v + c) & 7) * 16 + lane
                    tgt = jnp.where(act, tgt, dump)
                sidx[slot, pl.ds(v * 16, 16)] = tgt

            @pl.when(c >= 8)
            def _():
                pltpu.make_async_copy(
                    ones_v, hist_sh.at[sidx.at[slot]], sem).wait()
            pltpu.async_copy(
                ones_v, hist_sh.at[sidx.at[slot]], sem, add=True)

        @pl.loop(0, 8)
        def _(slot):
            pltpu.make_async_copy(
                ones_v, hist_sh.at[sidx.at[slot]], sem).wait()

        plsc.subcore_barrier()

        for dgt in range(16):
            pltpu.sync_copy(
                hist_sh.at[pl.ds(
                    pl.multiple_of(dgt * _NPAD + nbase, 8), _NODES_PW)],
                hstage.at[dgt])

        @pl.loop(0, _NV)
        def _(j):
            sl = pl.ds(j * 16, 16)
            if p == 0:
                deg = jnp.zeros((16,), jnp.int32)
                for dgt in range(16):
                    deg = deg + hstage[dgt, sl]
                keep = (deg + 1) >> 1
                need = keep
                invf_v[sl] = 1.0 / jnp.maximum(keep, 1).astype(jnp.float32)
            else:
                need = need_v[sl]
            cum = jnp.zeros((16,), jnp.int32)
            done = need <= 0
            sel = jnp.zeros((16,), jnp.int32)
            cumb = jnp.zeros((16,), jnp.int32)
            heq = jnp.ones((16,), jnp.int32)
            for dgt in range(16):
                h = hstage[dgt, sl]
                nc = cum + h
                cross = jnp.logical_and(jnp.logical_not(done), nc >= need)
                sel = jnp.where(cross, dgt, sel)
                cumb = jnp.where(cross, cum, cumb)
                heq = jnp.where(cross, h, heq)
                done = jnp.logical_or(done, cross)
                cum = nc
            newneed = need - cumb
            need_v[sl] = newneed
            if p == 0:
                newpref = sel
            else:
                oldpref = pref_v[pl.ds(nbase + j * 16, 16)]
                newpref = (oldpref << 4) | sel
            prefst_v[sl] = newpref
            if p == _NPASS - 1:
                frac = newneed.astype(jnp.float32) / heq.astype(jnp.float32)
                fracst_v[sl] = jnp.where(done, frac, 0.0)

        pltpu.sync_copy(
            prefst_v,
            pref_sh.at[pl.ds(pl.multiple_of(nbase, 8), _NODES_PW)])
        if p == 0:
            pltpu.sync_copy(
                invf_v,
                invc_out.at[pl.ds(pl.multiple_of(nbase, 8), _NODES_PW)])
        if p == _NPASS - 1:
            pltpu.sync_copy(
                fracst_v,
                frac_sh.at[pl.ds(pl.multiple_of(nbase, 8), _NODES_PW)])
            pltpu.sync_copy(
                fracst_v,
                frac_out.at[pl.ds(pl.multiple_of(nbase, 8), _NODES_PW)])
        plsc.subcore_barrier()
        pltpu.sync_copy(pref_sh, pref_v)
        if p == _NPASS - 1:
            pltpu.sync_copy(frac_sh, frac_v)

    @pl.loop(0, _NCH, init_carry=jnp.int32(0))
    def cnt_fin(c, cnt):
        for v in range(8):
            e = pl.ds(c * _CH + v * 16, 16)
            s = src_v[e]
            t = dst_v[e]
            k = key_v[e]
            thr = plsc.load_gather(pref_v, [t])
            f = plsc.load_gather(frac_v, [t])
            keepm = k <= thr
            isfr = jnp.logical_and(k == thr, f < 1.0)
            packed = s | (t << 14) | jnp.where(isfr, 1 << 28, 0)
            plsc.store_compressed(
                hw_v.at[pl.ds(cnt, 16)],
                plsc.bitcast(packed, jnp.float32), mask=keepm)
            cnt = cnt + plsc.all_reduce_population_count(keepm)[0]
        return cnt

    sent = plsc.bitcast((_NPAD + lane) << 14, jnp.float32)
    for i in range(_CCH // 16):
        hw_v[pl.ds(cnt_fin + i * 16, 16)] = sent
    cstage[pl.ds(0, 16)] = jnp.zeros((16,), jnp.int32) + cnt_fin
    ebase_c = pl.multiple_of(wid * _EPWC, 8)
    pltpu.sync_copy(hw_v, cpk_out.at[pl.ds(ebase_c, _EPWC)])
    pltpu.sync_copy(cstage.at[pl.ds(0, 8)],
                    cnts_out.at[pl.ds(pl.multiple_of(wid * 8, 8), 8)])


def _aggregate_body(cpk_hbm, x_hbm, invc_hbm, frac_hbm, cnts_hbm, acc_out,
                    pkr, sidxg, row_buf, idx_buf, invc_v, fracv, cntv,
                    acc_sh, stsem, gsem, ssem):
    wid = lax.axis_index("s")
    nbase = wid * _NODES_PW
    ebase = pl.multiple_of(wid * _EPWC, 8)
    lane = lax.iota(jnp.int32, 16)

    pltpu.sync_copy(
        cnts_hbm.at[pl.ds(pl.multiple_of(wid * 8, 8), 16)], cntv)
    cnt = cntv[pl.ds(0, 16)][0]
    nch = jnp.maximum((cnt + (_CCH - 1)) // _CCH, 1)
    pltpu.sync_copy(frac_hbm, fracv)

    @pl.loop(0, _CCH)
    def _(r):
        for q in range(8):
            row_buf[0, r, pl.ds(q * 16, 16)] = jnp.zeros((16,), jnp.float32)

    for b in range(6):
        pltpu.sync_copy(row_buf.at[0],
                        acc_sh.at[pl.ds(nbase + b * _CCH, _CCH)])
    pltpu.sync_copy(row_buf.at[0, pl.ds(0, 64)],
                    acc_sh.at[pl.ds(nbase + 576, 64)])

    @pl.when(wid == 0)
    def _():
        pltpu.sync_copy(row_buf.at[0, pl.ds(0, 16)],
                        acc_sh.at[pl.ds(_NPAD, 16)])
    plsc.subcore_barrier()

    def stage(c):
        r = lax.rem(c, 4)
        pltpu.async_copy(
            cpk_hbm.at[pl.ds(ebase + c * _CCH, _CCH)], pkr.at[r], stsem)

    def wait_stage(c):
        r = lax.rem(c, 4)
        pltpu.make_async_copy(
            cpk_hbm.at[pl.ds(ebase + c * _CCH, _CCH)], pkr.at[r],
            stsem).wait()

    def unpack(c):
        r = lax.rem(c, 4)
        b = lax.rem(c, 2)
        for v in range(_CCH // 16):
            e = pl.ds(v * 16, 16)
            pk = plsc.bitcast(pkr[r, e], jnp.int32)
            sidxg[b, e] = pk & 16383

    stage(0)
    stage(1)
    stage(2)
    wait_stage(0)
    unpack(0)
    pltpu.async_copy(x_hbm.at[sidxg.at[0]], row_buf.at[0], gsem)

    @pl.loop(0, nch)
    def _(c):
        b = lax.rem(c, 2)
        r = lax.rem(c, 4)

        @pl.when(c + 3 < nch)
        def _():
            stage(c + 3)

        @pl.when(c >= 1)
        def _():
            bo = lax.rem(c - 1, 2)
            pltpu.make_async_copy(
                row_buf.at[bo], acc_sh.at[idx_buf.at[bo]], ssem).wait()

        @pl.when(c + 1 < nch)
        def _():
            bn = lax.rem(c + 1, 2)
            wait_stage(c + 1)
            unpack(c + 1)
            pltpu.async_copy(x_hbm.at[sidxg.at[bn]], row_buf.at[bn], gsem)

        pltpu.make_async_copy(
            x_hbm.at[sidxg.at[b]], row_buf.at[b], gsem).wait()

        for v in range(_CCH // 16):
            e = pl.ds(v * 16, 16)
            pk = plsc.bitcast(pkr[r, e], jnp.int32)
            tgt = (pk >> 14) & 16383
            idx_buf[b, e] = tgt
            isfr = (pk >> 28) == 1
            nfrac = jnp.max(jnp.where(isfr, 1, 0))

            @pl.when(nfrac > 0)
            def _():
                fv = plsc.load_gather(fracv, [jnp.where(isfr, tgt, 0)])
                fv = jnp.where(isfr, fv, 1.0)

                @pl.loop(0, 16)
                def _(l):
                    wl = jnp.min(jnp.where(lane == l, fv, 2.0))

                    @pl.when(wl < 1.0)
                    def _():
                        for q in range(8):
                            row_buf[b, v * 16 + l, pl.ds(q * 16, 16)] = (
                                row_buf[b, v * 16 + l, pl.ds(q * 16, 16)]
                                * wl)

        pltpu.async_copy(
            row_buf.at[b], acc_sh.at[idx_buf.at[b]], ssem, add=True)

    bo = lax.rem(nch - 1, 2)
    pltpu.make_async_copy(
        row_buf.at[bo], acc_sh.at[idx_buf.at[bo]], ssem).wait()
    plsc.subcore_barrier()

    pltpu.sync_copy(
        invc_hbm.at[pl.ds(pl.multiple_of(nbase, 8), _NODES_PW)], invc_v)

    for b in range(7):
        rows = _CCH if b < 6 else 64
        start = b * _CCH
        pltpu.sync_copy(acc_sh.at[pl.ds(nbase + start, rows)],
                        row_buf.at[0, pl.ds(0, rows)])

        @pl.loop(0, rows // 16)
        def _(g):
            iv = invc_v[pl.ds(start + g * 16, 16)] * _P
            for r in range(16):
                s = iv[r]
                for q in range(8):
                    row_buf[0, g * 16 + r, pl.ds(q * 16, 16)] = (
                        row_buf[0, g * 16 + r, pl.ds(q * 16, 16)] * s)
        pltpu.sync_copy(
            row_buf.at[0, pl.ds(0, rows)],
            acc_out.at[pl.ds(pl.multiple_of(nbase + start, 8), rows)])


_sc_select = functools.partial(
    pl.kernel,
    out_type=[
        jax.ShapeDtypeStruct((_NW * _EPWC,), jnp.float32),
        jax.ShapeDtypeStruct((144,), jnp.int32),
        jax.ShapeDtypeStruct((_NPAD,), jnp.float32),
        jax.ShapeDtypeStruct((_NPAD,), jnp.float32),
    ],
    mesh=_mesh,
    compiler_params=_params,
    scratch_types=[
        pltpu.VMEM((_EPW,), jnp.int32),
        pltpu.VMEM((_EPW,), jnp.int32),
        pltpu.VMEM((_EPW,), jnp.int32),
        pltpu.VMEM((_EPWC,), jnp.float32),
        pltpu.VMEM((_NPAD,), jnp.int32),
        pltpu.VMEM((_NPAD,), jnp.float32),
        pltpu.VMEM((16, _NODES_PW), jnp.int32),
        pltpu.VMEM((_NODES_PW,), jnp.int32),
        pltpu.VMEM((_NODES_PW,), jnp.int32),
        pltpu.VMEM((_NODES_PW,), jnp.float32),
        pltpu.VMEM((_NODES_PW,), jnp.float32),
        pltpu.VMEM((_NODES_PW,), jnp.int32),
        pltpu.VMEM((_CH,), jnp.int32),
        pltpu.VMEM((8, _CH), jnp.int32),
        pltpu.VMEM((16,), jnp.int32),
        pltpu.VMEM_SHARED((_HIST + 128,), jnp.int32),
        pltpu.VMEM_SHARED((_NPAD,), jnp.int32),
        pltpu.VMEM_SHARED((_NPAD,), jnp.float32),
        pltpu.SemaphoreType.DMA,
    ],
)(_select_body)

_sc_aggregate = functools.partial(
    pl.kernel,
    out_type=[jax.ShapeDtypeStruct((_NPAD, _D), jnp.float32)],
    mesh=_mesh,
    compiler_params=_params,
    scratch_types=[
        pltpu.VMEM((4, _CCH), jnp.float32),
        pltpu.VMEM((2, _CCH), jnp.int32),
        pltpu.VMEM((2, _CCH, _D), jnp.float32),
        pltpu.VMEM((2, _CCH), jnp.int32),
        pltpu.VMEM((_NODES_PW,), jnp.float32),
        pltpu.VMEM((_NPAD,), jnp.float32),
        pltpu.VMEM((16,), jnp.int32),
        pltpu.VMEM_SHARED((_NPAD + 16, _D), jnp.float32),
        pltpu.SemaphoreType.DMA,
        pltpu.SemaphoreType.DMA,
        pltpu.SemaphoreType.DMA,
    ],
)(_aggregate_body)


def kernel(x, W_mlp, b_mlp, W_lin, b_lin, edge_index):
    x_pad = jnp.pad(x, ((0, _NPAD - _N), (0, 0)))
    w_big = jnp.zeros((_D, _D), jnp.float32).at[:, :2].set(W_mlp.T)
    b_big = jnp.zeros((_D,), jnp.float32).at[:2].set(b_mlp)
    h_full = _tc_mlp(x_pad, w_big, b_big[None, :])
    h_flat = h_full[:, :2].reshape(-1)

    src = edge_index[0]
    dst = edge_index[1]
    pad = _EPAD - _E
    src_p = jnp.concatenate([src, jnp.zeros((pad,), jnp.int32)])
    dst_p = jnp.concatenate([dst, jnp.full((pad,), _NPAD - 1, jnp.int32)])

    cpk, cnts, invc, fracn = _sc_select(src_p, dst_p, h_flat)
    (acc,) = _sc_aggregate(cpk, x, invc, fracn, cnts)
    out_pad = _tc_out(acc, x_pad, W_lin.T, b_lin[None, :])
    return out_pad[:_N]
